# baseline (speedup 1.0000x reference)
import functools
import math

import jax
import jax.numpy as jnp
from jax.experimental import pallas as pl
from jax.experimental.pallas import tpu as pltpu

_NZ = 100
_NL = 100000
_NU = 1024
_BK = 2048
_NBLK = (_NL + _BK - 1) // _BK
_LOG_NORM = 0.5 * _NZ * math.log(2.0 * math.pi)


def _dd_body(uT_ref, L_ref, o_ref, tmin_ref):
    i = pl.program_id(0)
    Lb = L_ref[...]
    uT = uT_ref[...]
    P = jax.lax.dot_general(
        Lb, uT, (((1,), (0,)), ((), ())),
        preferred_element_type=jnp.float32,
    )
    L2 = jnp.sum(Lb * Lb, axis=1, keepdims=True)
    t = L2 - 2.0 * P
    gidx = i * _BK + jax.lax.broadcasted_iota(jnp.int32, (_BK, 1), 0)
    t = jnp.where(gidx < _NL, t, jnp.inf)
    bmin = jnp.min(t, axis=0, keepdims=True)

    @pl.when(i == 0)
    def _():
        tmin_ref[...] = bmin

    @pl.when(i > 0)
    def _():
        tmin_ref[...] = jnp.minimum(tmin_ref[...], bmin)

    @pl.when(i == _NBLK - 1)
    def _():
        U2 = jnp.sum(uT * uT, axis=0, keepdims=True)
        d2 = jnp.maximum(tmin_ref[...] + U2, 0.0)
        div = jnp.log(jnp.sqrt(d2) + 1e-18)
        dens = -0.5 * U2 - _LOG_NORM
        dd = jnp.exp(dens + div)
        dd = dd - jnp.min(dd)
        o_ref[...] = dd / (jnp.max(dd) + 1e-18)


@functools.partial(jax.jit, static_argnames=("interpret",))
def _dd_call(uT, L_z, interpret=False):
    return pl.pallas_call(
        _dd_body,
        grid=(_NBLK,),
        in_specs=[
            pl.BlockSpec((_NZ, _NU), lambda i: (0, 0)),
            pl.BlockSpec((_BK, _NZ), lambda i: (i, 0)),
        ],
        out_specs=pl.BlockSpec((1, _NU), lambda i: (0, 0)),
        out_shape=jax.ShapeDtypeStruct((1, _NU), jnp.float32),
        scratch_shapes=[pltpu.VMEM((1, _NU), jnp.float32)],
        compiler_params=pltpu.CompilerParams(
            dimension_semantics=("arbitrary",),
        ),
        interpret=interpret,
    )(uT, L_z)


def kernel(pred, U_z, L_z):
    del pred
    out = _dd_call(U_z.T, L_z)
    return out.reshape(-1)


# speedup vs baseline: 1.0211x; 1.0211x over previous
import functools
import math

import jax
import jax.numpy as jnp
from jax.experimental import pallas as pl
from jax.experimental.pallas import tpu as pltpu

_NZ = 100
_NL = 100000
_NU = 1024
_BK = 2048
_NBLK = (_NL + _BK - 1) // _BK
_LOG_NORM = 0.5 * _NZ * math.log(2.0 * math.pi)


def _dd_body(uTm2_ref, L_ref, o_ref, tmin_ref):
    i = pl.program_id(0)
    Lb = L_ref[...]
    uTm2 = uTm2_ref[...]
    P = jax.lax.dot_general(
        Lb.astype(jnp.bfloat16), uTm2.astype(jnp.bfloat16),
        (((1,), (0,)), ((), ())),
        preferred_element_type=jnp.float32,
    )
    L2 = jnp.sum(Lb * Lb, axis=1, keepdims=True)
    t = L2 + P

    @pl.when(i < _NBLK - 1)
    def _():
        bmin = jnp.min(t, axis=0, keepdims=True)
        if _NBLK > 1:
            tmin_ref[...] = jnp.where(i == 0, bmin,
                                      jnp.minimum(tmin_ref[...], bmin))

    @pl.when(i == _NBLK - 1)
    def _():
        gidx = i * _BK + jax.lax.broadcasted_iota(jnp.int32, (_BK, 1), 0)
        bmin = jnp.min(jnp.where(gidx < _NL, t, jnp.inf),
                       axis=0, keepdims=True)
        tmin = jnp.minimum(tmin_ref[...], bmin)
        U2 = 0.25 * jnp.sum(uTm2 * uTm2, axis=0, keepdims=True)
        d2 = jnp.maximum(tmin + U2, 0.0)
        div = jnp.log(jnp.sqrt(d2) + 1e-18)
        dens = -0.5 * U2 - _LOG_NORM
        dd = jnp.exp(dens + div)
        dd = dd - jnp.min(dd)
        o_ref[...] = dd / (jnp.max(dd) + 1e-18)


@functools.partial(jax.jit, static_argnames=("interpret",))
def _dd_call(uT, L_z, interpret=False):
    return pl.pallas_call(
        _dd_body,
        grid=(_NBLK,),
        in_specs=[
            pl.BlockSpec((_NZ, _NU), lambda i: (0, 0)),
            pl.BlockSpec((_BK, _NZ), lambda i: (i, 0)),
        ],
        out_specs=pl.BlockSpec((1, _NU), lambda i: (0, 0)),
        out_shape=jax.ShapeDtypeStruct((1, _NU), jnp.float32),
        scratch_shapes=[pltpu.VMEM((1, _NU), jnp.float32)],
        compiler_params=pltpu.CompilerParams(
            dimension_semantics=("arbitrary",),
        ),
        interpret=interpret,
    )(uT, L_z)


def kernel(pred, U_z, L_z):
    del pred
    out = _dd_call(-2.0 * U_z.T, L_z)
    return out.reshape(-1)
